# baseline (speedup 1.0000x reference)
import functools

import jax
import jax.numpy as jnp
from jax import lax
from jax.experimental import pallas as pl
from jax.experimental.pallas import tpu as pltpu
from jax.experimental.pallas import tpu_sc as plsc

N_NODES = 10000
N_EDGES = 320000
D_FEAT = 128
D_EDGE = 16
HID = 128
OUTDIM = 10
NUM_GRAPHS = 64

NC = 2
NS = 16
NW = NC * NS
EK = 96
NBUF = 2
CPT_PAD = 108
NCHUNK_PAD = CPT_PAD * NW
N_EDGES_PAD = NCHUNK_PAD * EK
ROWS_MAIN = 632
ROWS_LAST = N_NODES - 15 * ROWS_MAIN

_sc_mesh = plsc.VectorSubcoreMesh(core_axis_name="c", subcore_axis_name="s")


@functools.partial(
    pl.kernel,
    mesh=_sc_mesh,
    out_type=jax.ShapeDtypeStruct((NC, N_NODES, HID), jnp.float32),
    scratch_types=[
        pltpu.VMEM((NBUF, EK), jnp.int32),
        pltpu.VMEM((NBUF, EK), jnp.int32),
        pltpu.VMEM((NBUF, EK, HID), jnp.float32),
        pltpu.VMEM((NBUF, EK, HID), jnp.float32),
        pltpu.VMEM_SHARED((N_NODES, HID), jnp.float32),
        pltpu.SemaphoreType.DMA,
        pltpu.SemaphoreType.DMA,
        pltpu.SemaphoreType.DMA,
        pltpu.SemaphoreType.DMA,
    ],
)
def _edge_agg_sc(pre_hbm, eap_hbm, src_hbm, dst_hbm, zeros_hbm, out_hbm,
                 src_i, dst_i, rows_v, eap_v, acc_sh, g0, g1, i0, i1):
    cid = lax.axis_index("c")
    sid = lax.axis_index("s")
    wid = sid * NC + cid
    gsems = (g0, g1)
    isems = (i0, i1)

    row0 = pl.multiple_of(sid * ROWS_MAIN, 8)

    @pl.when(sid < NS - 1)
    def _():
        pltpu.sync_copy(zeros_hbm.at[pl.ds(row0, ROWS_MAIN)],
                        acc_sh.at[pl.ds(row0, ROWS_MAIN)])

    @pl.when(sid == NS - 1)
    def _():
        pltpu.sync_copy(zeros_hbm.at[pl.ds(row0, ROWS_LAST)],
                        acc_sh.at[pl.ds(row0, ROWS_LAST)])

    plsc.subcore_barrier()

    def ebase(j):
        return pl.multiple_of((j * NW + wid) * EK, 32)

    def issue_streams(j, b):
        pltpu.async_copy(pre_hbm.at[src_i.at[b]], rows_v.at[b], gsems[b])
        pltpu.async_copy(eap_hbm.at[pl.ds(ebase(j), EK)], eap_v.at[b],
                         gsems[b])

    pltpu.sync_copy(src_hbm.at[pl.ds(ebase(0), EK)], src_i.at[0])
    pltpu.sync_copy(dst_hbm.at[pl.ds(ebase(0), EK)], dst_i.at[0])
    issue_streams(0, 0)
    pltpu.async_copy(src_hbm.at[pl.ds(ebase(1), EK)], src_i.at[1], isems[1])
    pltpu.async_copy(dst_hbm.at[pl.ds(ebase(1), EK)], dst_i.at[1], isems[1])

    NG = CPT_PAD // NBUF

    def group_body(g, carry):
        for b in range(NBUF):
            j = g * NBUF + b
            nb = 1 - b

            def start_next():
                pltpu.make_async_copy(src_hbm.at[pl.ds(0, EK)],
                                      src_i.at[nb], isems[nb]).wait()
                pltpu.make_async_copy(dst_hbm.at[pl.ds(0, EK)],
                                      dst_i.at[nb], isems[nb]).wait()
                issue_streams(j + 1, nb)
            if b == 0:
                start_next()
            else:
                pl.when(g < NG - 1)(start_next)

            pltpu.make_async_copy(pre_hbm.at[src_i.at[b]], rows_v.at[b],
                                  gsems[b]).wait()
            pltpu.make_async_copy(eap_hbm.at[pl.ds(0, EK)], eap_v.at[b],
                                  gsems[b]).wait()

            @pl.when(g < NG - 1)
            def _():
                pltpu.async_copy(src_hbm.at[pl.ds(ebase(j + 2), EK)],
                                 src_i.at[b], isems[b])

            def row_body(i, c2):
                for c in range(HID // 16):
                    sl = pl.ds(c * 16, 16)
                    eap_v[b, i, sl] = jnp.maximum(
                        rows_v[b, i, sl] + eap_v[b, i, sl], 0.0)
                return c2
            lax.fori_loop(0, EK, row_body, 0)

            pltpu.sync_copy(eap_v.at[b], acc_sh.at[dst_i.at[b]], add=True)

            @pl.when(g < NG - 1)
            def _():
                pltpu.async_copy(dst_hbm.at[pl.ds(ebase(j + 2), EK)],
                                 dst_i.at[b], isems[b])
        return carry

    lax.fori_loop(0, NG, group_body, 0)
    plsc.subcore_barrier()

    @pl.when(sid < NS - 1)
    def _():
        pltpu.sync_copy(acc_sh.at[pl.ds(row0, ROWS_MAIN)],
                        out_hbm.at[cid, pl.ds(row0, ROWS_MAIN)])

    @pl.when(sid == NS - 1)
    def _():
        pltpu.sync_copy(acc_sh.at[pl.ds(row0, ROWS_LAST)],
                        out_hbm.at[cid, pl.ds(row0, ROWS_LAST)])


def _pre_body(h_ref, w_ref, o_ref):
    o_ref[...] = jnp.dot(h_ref[...], w_ref[...],
                         preferred_element_type=jnp.float32)


def _pre_tc(h, w1a):
    return pl.pallas_call(
        _pre_body,
        out_shape=jax.ShapeDtypeStruct((N_NODES, HID), jnp.float32),
    )(h, w1a)


DBLK = 4000
NHI = 80


def _deg_body(dst_ref, o_ref):
    d = dst_ref[...]
    hi = lax.shift_right_logical(d, 7)
    lo = lax.bitwise_and(d, 127)
    oh_hi = (hi == lax.broadcasted_iota(jnp.int32, (1, NHI), 1)
             ).astype(jnp.float32)
    oh_lo = (lo == lax.broadcasted_iota(jnp.int32, (1, HID), 1)
             ).astype(jnp.float32)
    c = lax.dot_general(oh_hi, oh_lo, (((0,), (0,)), ((), ())),
                        preferred_element_type=jnp.float32)

    @pl.when(pl.program_id(0) == 0)
    def _():
        o_ref[...] = c

    @pl.when(pl.program_id(0) != 0)
    def _():
        o_ref[...] += c


def _degree_tc(dst2d):
    return pl.pallas_call(
        _deg_body,
        grid=(N_EDGES // DBLK,),
        in_specs=[pl.BlockSpec((DBLK, 1), lambda i: (i, 0))],
        out_specs=pl.BlockSpec((NHI, HID), lambda i: (0, 0)),
        out_shape=jax.ShapeDtypeStruct((NHI, HID), jnp.float32),
    )(dst2d)


EBLK = 4096


def _eap_body(ea_ref, w_ref, b_ref, o_ref):
    val = (jnp.dot(ea_ref[...], w_ref[...],
                   preferred_element_type=jnp.float32)
           + b_ref[...])
    row = (pl.program_id(0) * EBLK
           + lax.broadcasted_iota(jnp.int32, (EBLK, 1), 0))
    o_ref[...] = jnp.where(row < N_EDGES, val, -1e30)


def _eap_tc(ea_pad, w1b, b1):
    return pl.pallas_call(
        _eap_body,
        grid=(N_EDGES_PAD // EBLK,),
        in_specs=[
            pl.BlockSpec((EBLK, D_EDGE), lambda i: (i, 0)),
            pl.BlockSpec((D_EDGE, HID), lambda i: (0, 0)),
            pl.BlockSpec((1, HID), lambda i: (0, 0)),
        ],
        out_specs=pl.BlockSpec((EBLK, HID), lambda i: (i, 0)),
        out_shape=jax.ShapeDtypeStruct((N_EDGES_PAD, HID), jnp.float32),
    )(ea_pad, w1b, b1)


def _node_body(h_ref, a0_ref, a1_ref, cnt_ref, w2_ref, b2_ref, wr_ref,
               br_ref, pa_ref, g_ref, beta_ref, o_ref):
    seg = a0_ref[...] + a1_ref[...]
    cnt = cnt_ref[...]
    aggm = jnp.dot(seg, w2_ref[...], preferred_element_type=jnp.float32)
    aggm = (aggm + cnt * b2_ref[...]) / jnp.maximum(cnt, 1.0)
    hh = (jnp.dot(h_ref[...], wr_ref[...], preferred_element_type=jnp.float32)
          + br_ref[...] + aggm)
    a = pa_ref[0, 0]
    hh = jnp.where(hh >= 0, hh, a * hh)
    mu = jnp.mean(hh, axis=0, keepdims=True)
    var = jnp.mean((hh - mu) ** 2, axis=0, keepdims=True)
    o_ref[...] = (hh - mu) * lax.rsqrt(var + 1e-5) * g_ref[...] + beta_ref[...]


def _node_tc(h, a0, a1, cnt2d, w2, b2, wr, br, pa, g, beta):
    return pl.pallas_call(
        _node_body,
        out_shape=jax.ShapeDtypeStruct((N_NODES, HID), jnp.float32),
    )(h, a0, a1, cnt2d, w2, b2, wr, br, pa, g, beta)


def _head_body(h_ref, b_ref, wh1_ref, bh1_ref, wh2_ref, bh2_ref, o_ref):
    batch = b_ref[...]
    gids = lax.broadcasted_iota(jnp.int32, (1, NUM_GRAPHS), 1)
    onehot = (batch == gids).astype(jnp.float32)
    psum = lax.dot_general(onehot, h_ref[...], (((0,), (0,)), ((), ())),
                           preferred_element_type=jnp.float32)
    ones = jnp.ones((N_NODES, 1), jnp.float32)
    pcnt = lax.dot_general(onehot, ones, (((0,), (0,)), ((), ())),
                           preferred_element_type=jnp.float32)
    pooled = psum / jnp.maximum(pcnt, 1.0)
    z = jnp.maximum(
        jnp.dot(pooled, wh1_ref[...], preferred_element_type=jnp.float32)
        + bh1_ref[...], 0.0)
    o_ref[...] = (jnp.dot(z, wh2_ref[...], preferred_element_type=jnp.float32)
                  + bh2_ref[...])


def _head_tc(h, batch2d, wh1, bh1, wh2, bh2):
    return pl.pallas_call(
        _head_body,
        out_shape=jax.ShapeDtypeStruct((NUM_GRAPHS, OUTDIM), jnp.float32),
    )(h, batch2d, wh1, bh1, wh2, bh2)


def kernel(x, edge_index, edge_attr, batch, W1s, b1s, W2s, b2s, Wrs, brs,
           prelu_a, gammas, betas, Wh1, bh1, Wh2, bh2):
    src = edge_index[0].astype(jnp.int32)
    dst = edge_index[1].astype(jnp.int32)
    npad = N_EDGES_PAD - N_EDGES
    pad_i = jnp.zeros((npad,), jnp.int32)
    src_p = jnp.concatenate([src, pad_i])
    dst_p = jnp.concatenate([dst, pad_i])
    ea_p = jnp.concatenate(
        [edge_attr, jnp.zeros((npad, D_EDGE), jnp.float32)])
    zeros_nh = jnp.zeros((N_NODES, HID), jnp.float32)

    degc = _degree_tc(dst.reshape(N_EDGES, 1))
    cnt2d = degc.reshape(NHI * HID)[:N_NODES].reshape(N_NODES, 1)

    h = x
    for i in range(4):
        w1a = W1s[i, :D_FEAT]
        w1b = W1s[i, D_FEAT:]
        pre = _pre_tc(h, w1a)
        eap = _eap_tc(ea_p, w1b, b1s[i].reshape(1, HID))
        parts = _edge_agg_sc(pre, eap, src_p, dst_p, zeros_nh)
        h = _node_tc(h, parts[0], parts[1], cnt2d, W2s[i],
                     b2s[i].reshape(1, HID), Wrs[i], brs[i].reshape(1, HID),
                     prelu_a[i].reshape(1, 1), gammas[i].reshape(1, HID),
                     betas[i].reshape(1, HID))

    return _head_tc(h, batch.astype(jnp.int32).reshape(N_NODES, 1),
                    Wh1, bh1.reshape(1, HID), Wh2, bh2.reshape(1, OUTDIM))
